# baseline (speedup 1.0000x reference)
import jax
import jax.numpy as jnp
from jax.experimental import pallas as pl


def _linear3(ws, x):
    for (W, b) in ws[:-1]:
        x = jax.nn.leaky_relu(x @ W + b, negative_slope=0.01)
    W, b = ws[-1]
    return x @ W + b


def _identity_kernel(x_ref, o_ref):
    o_ref[...] = x_ref[...]


def kernel(node_vectors, node_vectors_initial, u_indices, v_indices, edge_vectors, params):
    nv = node_vectors
    nvi = node_vectors_initial
    u = u_indices
    v = v_indices
    ev = edge_vectors[:, 0]
    mask0 = (ev == 0).astype(nv.dtype)[:, None]
    mask1 = (ev == 1).astype(nv.dtype)[:, None]
    N, D = nv.shape
    for i in range(2):
        agg = jnp.zeros((N, D), nv.dtype)
        msg = _linear3(params["f_ef"][i], jnp.concatenate([nv[u], nv[v], nvi[u], nvi[v]], axis=1))
        agg = agg.at[u].add(msg * mask0)
        msg = _linear3(params["f_ef"][i], jnp.concatenate([nv[v], nv[u], nvi[v], nvi[u]], axis=1))
        agg = agg.at[v].add(msg * mask0)
        nv = _linear3(params["f_n"][i], jnp.concatenate([agg, nv], axis=1))
        agg = jnp.zeros((N, D), nv.dtype)
        msg = _linear3(params["f_ef2"][i], jnp.concatenate([nv[u], nv[v], nvi[u], nvi[v]], axis=1))
        agg = agg.at[u].add(msg * mask1)
        msg = _linear3(params["f_ef2"][i], jnp.concatenate([nv[v], nv[u], nvi[v], nvi[u]], axis=1))
        agg = agg.at[v].add(msg * mask1)
        nv = _linear3(params["f_n2"][i], jnp.concatenate([agg, nv], axis=1))
    return pl.pallas_call(
        _identity_kernel,
        out_shape=jax.ShapeDtypeStruct(nv.shape, nv.dtype),
    )(nv)


# speedup vs baseline: 1.2330x; 1.2330x over previous
import functools

import jax
import jax.numpy as jnp
from jax import lax
from jax.experimental import pallas as pl
from jax.experimental.pallas import tpu as pltpu
from jax.experimental.pallas import tpu_sc as plsc

NC = 2
NS = 16
CH = 128
NB = 4

_SC_MESH = dict(core_axis_name="c", subcore_axis_name="s", num_cores=NC,
                num_subcores=NS)


def _leaky(x):
    return jnp.where(x > 0, x, 0.01 * x)


@functools.lru_cache(maxsize=None)
def _gather_pair(NPAD, EP, D):
    nchunks = EP // CH
    cpw = nchunks // (NC * NS)

    @functools.partial(
        pl.kernel,
        out_type=[jax.ShapeDtypeStruct((EP, D), jnp.float32),
                  jax.ShapeDtypeStruct((EP, D), jnp.float32)],
        mesh=plsc.VectorSubcoreMesh(**_SC_MESH),
        scratch_types=[pltpu.VMEM((cpw, CH), jnp.int32),
                       pltpu.VMEM((cpw, CH), jnp.int32),
                       pltpu.VMEM((NB, CH, D), jnp.float32),
                       pltpu.SemaphoreType.DMA,
                       pltpu.SemaphoreType.DMA,
                       pltpu.SemaphoreType.DMA,
                       pltpu.SemaphoreType.DMA],
    )
    def gather(table, u_idx, v_idx, out_u, out_v, ui_v, vi_v, buf,
               s0, s1, s2, s3):
        sems = (s0, s1, s2, s3)
        wid = lax.axis_index("s") * NC + lax.axis_index("c")
        base = wid * cpw
        pltpu.sync_copy(u_idx.at[pl.ds(base, cpw)], ui_v)
        pltpu.sync_copy(v_idx.at[pl.ds(base, cpw)], vi_v)
        for idx_v, out in ((ui_v, out_u), (vi_v, out_v)):
            for b in range(NB):
                pltpu.async_copy(table.at[idx_v.at[b]], buf.at[b], sems[b])

            def body(i, _, idx_v=idx_v, out=out):
                for b in range(NB):
                    j = i * NB + b
                    pltpu.make_async_copy(table.at[idx_v.at[j]], buf.at[b],
                                          sems[b]).wait()
                    pltpu.sync_copy(buf.at[b],
                                    out.at[pl.ds((base + j) * CH, CH)])

                    @pl.when(j + NB < cpw)
                    def _fire(idx_v=idx_v, b=b, j=j):
                        pltpu.async_copy(table.at[idx_v.at[j + NB]],
                                         buf.at[b], sems[b])
                return 0

            lax.fori_loop(0, cpw // NB, body, 0)

    return gather


@functools.lru_cache(maxsize=None)
def _scatter_pair(NPAD, EP, D):
    nchunks = EP // CH
    cps = nchunks // NC
    cpw = cps // NS
    rpt = NPAD // NS

    @functools.partial(
        pl.kernel,
        out_type=jax.ShapeDtypeStruct((NC, NPAD, D), jnp.float32),
        mesh=plsc.VectorSubcoreMesh(**_SC_MESH),
        scratch_types=[pltpu.VMEM((2, CH), jnp.int32),
                       pltpu.VMEM((2, CH, D), jnp.float32),
                       pltpu.VMEM_SHARED((NPAD, D), jnp.float32),
                       pltpu.SemaphoreType.DMA,
                       pltpu.SemaphoreType.DMA],
    )
    def scatter(msg_f, msg_r, u_idx, v_idx, zeros, out, ibuf, mbuf,
                agg, sA, sB):
        sems = (sA, sB)
        c = lax.axis_index("c")
        s = lax.axis_index("s")
        pltpu.sync_copy(zeros, agg.at[pl.ds(s * rpt, rpt)])
        base = c * cps + s * cpw
        plsc.subcore_barrier()
        for msg, iv in ((msg_f, u_idx), (msg_r, v_idx)):
            for b in range(2):
                pltpu.async_copy(msg.at[pl.ds((base + b) * CH, CH)],
                                 mbuf.at[b], sems[b])
                pltpu.async_copy(iv.at[base + b], ibuf.at[b], sems[b])

            def body(i, _, msg=msg, iv=iv):
                for b in range(2):
                    j = i * 2 + b
                    pltpu.make_async_copy(
                        msg.at[pl.ds((base + j) * CH, CH)], mbuf.at[b],
                        sems[b]).wait()
                    pltpu.make_async_copy(
                        iv.at[base + j], ibuf.at[b], sems[b]).wait()
                    pltpu.sync_copy(mbuf.at[b], agg.at[ibuf.at[b]], add=True)

                    @pl.when(j + 2 < cpw)
                    def _fire(msg=msg, iv=iv, b=b, j=j):
                        pltpu.async_copy(
                            msg.at[pl.ds((base + j + 2) * CH, CH)],
                            mbuf.at[b], sems[b])
                        pltpu.async_copy(iv.at[base + j + 2], ibuf.at[b],
                                         sems[b])
                return 0

            lax.fori_loop(0, cpw // 2, body, 0)
        plsc.subcore_barrier()
        pltpu.sync_copy(agg.at[pl.ds(s * rpt, rpt)],
                        out.at[c].at[pl.ds(s * rpt, rpt)])

    return scatter


def _edge_mlp_body(xu, xv, yu, yv, m, w1, b1, w2, b2, w3, b3, of, orv):
    D = xu.shape[-1]
    a, bb = xu[...], xv[...]
    cu, cv = yu[...], yv[...]
    W1 = w1[...]
    w1a = W1[0 * D:1 * D]
    w1b = W1[1 * D:2 * D]
    w1c = W1[2 * D:3 * D]
    w1d = W1[3 * D:4 * D]
    dot = functools.partial(jnp.dot, preferred_element_type=jnp.float32)
    mk = m[...]
    for (p, q, r, t, o) in ((a, bb, cu, cv, of), (bb, a, cv, cu, orv)):
        h = _leaky(dot(p, w1a) + dot(q, w1b) + dot(r, w1c) + dot(t, w1d)
                   + b1[...])
        h = _leaky(dot(h, w2[...]) + b2[...])
        o[...] = (dot(h, w3[...]) + b3[...]) * mk


@functools.lru_cache(maxsize=None)
def _edge_mlp(EP, D, H, B):
    grid = EP // B
    row = lambda i: (i, 0)
    full = lambda i: (0, 0)

    return pl.pallas_call(
        _edge_mlp_body,
        grid=grid,
        in_specs=[pl.BlockSpec((B, D), row)] * 4
        + [pl.BlockSpec((B, 1), row),
           pl.BlockSpec((4 * D, H), full), pl.BlockSpec((1, H), full),
           pl.BlockSpec((H, H), full), pl.BlockSpec((1, H), full),
           pl.BlockSpec((H, D), full), pl.BlockSpec((1, D), full)],
        out_specs=[pl.BlockSpec((B, D), row), pl.BlockSpec((B, D), row)],
        out_shape=[jax.ShapeDtypeStruct((EP, D), jnp.float32),
                   jax.ShapeDtypeStruct((EP, D), jnp.float32)],
    )


def _node_mlp_body(a0, a1, nv, w1, b1, w2, b2, w3, b3, o):
    D = nv.shape[-1]
    agg = a0[...] + a1[...]
    x = nv[...]
    dot = functools.partial(jnp.dot, preferred_element_type=jnp.float32)
    W1 = w1[...]
    h = _leaky(dot(agg, W1[0:D]) + dot(x, W1[D:2 * D]) + b1[...])
    h = _leaky(dot(h, w2[...]) + b2[...])
    o[...] = dot(h, w3[...]) + b3[...]


@functools.lru_cache(maxsize=None)
def _node_mlp(NPAD, D, H, B):
    grid = NPAD // B
    row = lambda i: (i, 0)
    full = lambda i: (0, 0)

    return pl.pallas_call(
        _node_mlp_body,
        grid=grid,
        in_specs=[pl.BlockSpec((B, D), row)] * 3
        + [pl.BlockSpec((2 * D, H), full), pl.BlockSpec((1, H), full),
           pl.BlockSpec((H, H), full), pl.BlockSpec((1, H), full),
           pl.BlockSpec((H, D), full), pl.BlockSpec((1, D), full)],
        out_specs=pl.BlockSpec((B, D), row),
        out_shape=jax.ShapeDtypeStruct((NPAD, D), jnp.float32),
    )


def kernel(node_vectors, node_vectors_initial, u_indices, v_indices,
           edge_vectors, params):
    N, D = node_vectors.shape
    E = u_indices.shape[0]
    align_e = NC * NS * CH * NB
    EP = ((E + align_e - 1) // align_e) * align_e
    align_n = NS * CH
    NPAD = ((N + align_n - 1) // align_n) * align_n

    u32 = jnp.pad(u_indices.astype(jnp.int32), (0, EP - E))
    v32 = jnp.pad(v_indices.astype(jnp.int32), (0, EP - E))
    u_r = u32.reshape(EP // CH, CH)
    v_r = v32.reshape(EP // CH, CH)
    ev = edge_vectors[:, 0]
    f32 = jnp.float32
    m0 = jnp.pad((ev == 0).astype(f32), (0, EP - E)).reshape(EP, 1)
    m1 = jnp.pad((ev == 1).astype(f32), (0, EP - E)).reshape(EP, 1)

    nv = jnp.pad(node_vectors, ((0, NPAD - N), (0, 0)))
    nvi = jnp.pad(node_vectors_initial, ((0, NPAD - N), (0, 0)))
    zeros = jnp.zeros((NPAD // NS, D), f32)

    gather = _gather_pair(NPAD, EP, D)
    scatter = _scatter_pair(NPAD, EP, D)
    emlp = _edge_mlp(EP, D, 4 * D, 1024)
    nmlp = _node_mlp(NPAD, D, 2 * D, 1024)

    def wflat(ws):
        out = []
        for (W, b) in ws:
            out.append(W)
            out.append(b.reshape(1, -1))
        return out

    yu, yv = gather(nvi, u_r, v_r)

    for i in range(len(params["f_n"])):
        for ename, nname, m in (("f_ef", "f_n", m0), ("f_ef2", "f_n2", m1)):
            xu, xv = gather(nv, u_r, v_r)
            mf, mr = emlp(xu, xv, yu, yv, m, *wflat(params[ename][i]))
            aggp = scatter(mf, mr, u_r, v_r, zeros)
            nv = nmlp(aggp[0], aggp[1], nv, *wflat(params[nname][i]))

    return nv[:N]


# speedup vs baseline: 1.6397x; 1.3299x over previous
import functools

import jax
import jax.numpy as jnp
from jax import lax
from jax.experimental import pallas as pl
from jax.experimental.pallas import tpu as pltpu
from jax.experimental.pallas import tpu_sc as plsc

NC = 2
NS = 16
NW = NC * NS
CH = 128
NB = 4

_SC_MESH = dict(core_axis_name="c", subcore_axis_name="s", num_cores=NC,
                num_subcores=NS)


def _leaky(x):
    return jnp.where(x > 0, x, 0.01 * x)


def _limit(ref):
    return ref[pl.ds(0, 16)][0]


def _slots(cl, wid):
    return lax.shift_right_logical(cl - wid + NW - 1, 5)


@functools.lru_cache(maxsize=None)
def _gather_pair(NPAD, EP, D):
    nchunks = EP // CH
    cpw = nchunks // NW

    @functools.partial(
        pl.kernel,
        out_type=[jax.ShapeDtypeStruct((EP, D), jnp.float32),
                  jax.ShapeDtypeStruct((EP, D), jnp.float32)],
        mesh=plsc.VectorSubcoreMesh(**_SC_MESH),
        scratch_types=[pltpu.VMEM((cpw, CH), jnp.int32),
                       pltpu.VMEM((cpw, CH), jnp.int32),
                       pltpu.VMEM((16,), jnp.int32),
                       pltpu.VMEM((NB, CH, D), jnp.float32),
                       pltpu.SemaphoreType.DMA,
                       pltpu.SemaphoreType.DMA,
                       pltpu.SemaphoreType.DMA,
                       pltpu.SemaphoreType.DMA],
    )
    def gather(table, u_str, v_str, clim, out_u, out_v, ui_v, vi_v, cl_v,
               buf, s0, s1, s2, s3):
        sems = (s0, s1, s2, s3)
        wid = lax.axis_index("s") * NC + lax.axis_index("c")
        pltpu.sync_copy(clim, cl_v)
        pltpu.sync_copy(u_str.at[wid], ui_v)
        pltpu.sync_copy(v_str.at[wid], vi_v)
        cl = _limit(cl_v)
        nk = _slots(cl, wid)
        for idx_v, out in ((ui_v, out_u), (vi_v, out_v)):
            for b in range(NB):
                @pl.when(b < nk)
                def _prime(idx_v=idx_v, b=b):
                    pltpu.async_copy(table.at[idx_v.at[b]], buf.at[b],
                                     sems[b])

            def body(i, _, idx_v=idx_v, out=out):
                for b in range(NB):
                    j = i * NB + b

                    @pl.when(j < nk)
                    def _step(idx_v=idx_v, out=out, b=b, j=j):
                        pltpu.make_async_copy(table.at[idx_v.at[j]],
                                              buf.at[b], sems[b]).wait()
                        chunk = wid + j * NW
                        pltpu.sync_copy(buf.at[b],
                                        out.at[pl.ds(chunk * CH, CH)])

                        @pl.when(j + NB < nk)
                        def _fire(idx_v=idx_v, b=b, j=j):
                            pltpu.async_copy(table.at[idx_v.at[j + NB]],
                                             buf.at[b], sems[b])
                return 0

            lax.fori_loop(0, cpw // NB, body, 0)

    return gather


@functools.lru_cache(maxsize=None)
def _scatter_pair(NPAD, EP, D):
    nchunks = EP // CH
    cpw = nchunks // NW
    rpt = NPAD // NS

    @functools.partial(
        pl.kernel,
        out_type=jax.ShapeDtypeStruct((NC, NPAD, D), jnp.float32),
        mesh=plsc.VectorSubcoreMesh(**_SC_MESH),
        scratch_types=[pltpu.VMEM((2, CH), jnp.int32),
                       pltpu.VMEM((16,), jnp.int32),
                       pltpu.VMEM((2, CH, D), jnp.float32),
                       pltpu.VMEM_SHARED((NPAD, D), jnp.float32),
                       pltpu.SemaphoreType.DMA,
                       pltpu.SemaphoreType.DMA],
    )
    def scatter(msg_f, msg_r, u_idx, v_idx, clim, zeros, out, ibuf, cl_v,
                mbuf, agg, sA, sB):
        sems = (sA, sB)
        c = lax.axis_index("c")
        s = lax.axis_index("s")
        wid = s * NC + c
        pltpu.sync_copy(clim, cl_v)
        pltpu.sync_copy(zeros, agg.at[pl.ds(s * rpt, rpt)])
        cl = _limit(cl_v)
        nk = _slots(cl, wid)
        plsc.subcore_barrier()
        for msg, iv in ((msg_f, u_idx), (msg_r, v_idx)):
            for b in range(2):
                @pl.when(b < nk)
                def _prime(msg=msg, iv=iv, b=b):
                    chunk = wid + b * NW
                    pltpu.async_copy(msg.at[pl.ds(chunk * CH, CH)],
                                     mbuf.at[b], sems[b])
                    pltpu.async_copy(iv.at[chunk], ibuf.at[b], sems[b])

            def body(i, _, msg=msg, iv=iv):
                for b in range(2):
                    j = i * 2 + b

                    @pl.when(j < nk)
                    def _step(msg=msg, iv=iv, b=b, j=j):
                        chunk = wid + j * NW
                        pltpu.make_async_copy(
                            msg.at[pl.ds(chunk * CH, CH)], mbuf.at[b],
                            sems[b]).wait()
                        pltpu.make_async_copy(
                            iv.at[chunk], ibuf.at[b], sems[b]).wait()
                        pltpu.sync_copy(mbuf.at[b], agg.at[ibuf.at[b]],
                                        add=True)

                        @pl.when(j + 2 < nk)
                        def _fire(msg=msg, iv=iv, b=b, j=j):
                            nchunk = wid + (j + 2) * NW
                            pltpu.async_copy(
                                msg.at[pl.ds(nchunk * CH, CH)],
                                mbuf.at[b], sems[b])
                            pltpu.async_copy(iv.at[nchunk], ibuf.at[b],
                                             sems[b])
                return 0

            lax.fori_loop(0, cpw // 2, body, 0)
        plsc.subcore_barrier()
        pltpu.sync_copy(agg.at[pl.ds(s * rpt, rpt)],
                        out.at[c].at[pl.ds(s * rpt, rpt)])

    return scatter


def _edge_mlp_body(cnt, xu, xv, yu, yv, m, w1, b1, w2, b2, w3, b3, of, orv):
    i = pl.program_id(0)
    B = xu.shape[0]

    @pl.when(i * B < cnt[0])
    def _go():
        D = xu.shape[-1]
        a, bb = xu[...], xv[...]
        cu, cv = yu[...], yv[...]
        W1 = w1[...]
        w1a = W1[0 * D:1 * D]
        w1b = W1[1 * D:2 * D]
        w1c = W1[2 * D:3 * D]
        w1d = W1[3 * D:4 * D]
        dot = functools.partial(jnp.dot, preferred_element_type=jnp.float32)
        mk = m[...]
        for (p, q, r, t, o) in ((a, bb, cu, cv, of), (bb, a, cv, cu, orv)):
            h = _leaky(dot(p, w1a) + dot(q, w1b) + dot(r, w1c) + dot(t, w1d)
                       + b1[...])
            h = _leaky(dot(h, w2[...]) + b2[...])
            o[...] = (dot(h, w3[...]) + b3[...]) * mk


@functools.lru_cache(maxsize=None)
def _edge_mlp(EP, D, H, B):
    grid = EP // B

    def row(i, cnt):
        return (jnp.where(i * B < cnt[0], i, grid - 1), 0)

    def full(i, cnt):
        return (0, 0)

    gs = pltpu.PrefetchScalarGridSpec(
        num_scalar_prefetch=1,
        grid=(grid,),
        in_specs=[pl.BlockSpec((B, D), row)] * 4
        + [pl.BlockSpec((B, 1), row),
           pl.BlockSpec((4 * D, H), full), pl.BlockSpec((1, H), full),
           pl.BlockSpec((H, H), full), pl.BlockSpec((1, H), full),
           pl.BlockSpec((H, D), full), pl.BlockSpec((1, D), full)],
        out_specs=[pl.BlockSpec((B, D), row), pl.BlockSpec((B, D), row)],
    )
    return pl.pallas_call(
        _edge_mlp_body,
        grid_spec=gs,
        out_shape=[jax.ShapeDtypeStruct((EP, D), jnp.float32),
                   jax.ShapeDtypeStruct((EP, D), jnp.float32)],
    )


def _node_mlp_body(a0, a1, nv, w1, b1, w2, b2, w3, b3, o):
    D = nv.shape[-1]
    agg = a0[...] + a1[...]
    x = nv[...]
    dot = functools.partial(jnp.dot, preferred_element_type=jnp.float32)
    W1 = w1[...]
    h = _leaky(dot(agg, W1[0:D]) + dot(x, W1[D:2 * D]) + b1[...])
    h = _leaky(dot(h, w2[...]) + b2[...])
    o[...] = dot(h, w3[...]) + b3[...]


@functools.lru_cache(maxsize=None)
def _node_mlp(NPAD, D, H, B):
    grid = NPAD // B
    row = lambda i: (i, 0)
    full = lambda i: (0, 0)

    return pl.pallas_call(
        _node_mlp_body,
        grid=grid,
        in_specs=[pl.BlockSpec((B, D), row)] * 3
        + [pl.BlockSpec((2 * D, H), full), pl.BlockSpec((1, H), full),
           pl.BlockSpec((H, H), full), pl.BlockSpec((1, H), full),
           pl.BlockSpec((H, D), full), pl.BlockSpec((1, D), full)],
        out_specs=pl.BlockSpec((B, D), row),
        out_shape=jax.ShapeDtypeStruct((NPAD, D), jnp.float32),
    )


def _stride_chunks(idx_flat, EP):
    return (idx_flat.reshape(EP // CH // NW, NW, CH).transpose(1, 0, 2))


def kernel(node_vectors, node_vectors_initial, u_indices, v_indices,
           edge_vectors, params):
    N, D = node_vectors.shape
    E = u_indices.shape[0]
    align_e = NW * CH * NB
    EP = ((E + 1024 + align_e - 1) // align_e) * align_e
    align_n = NS * CH
    NPAD = ((N + align_n - 1) // align_n) * align_n
    f32 = jnp.float32
    i32 = jnp.int32

    u32 = u_indices.astype(i32)
    v32 = v_indices.astype(i32)
    ev = edge_vectors[:, 0]

    is0 = (ev == 0).astype(i32)
    c0 = jnp.sum(is0)
    c1 = E - c0
    p0 = jnp.cumsum(is0) - 1
    p1 = jnp.cumsum(1 - is0) - 1
    eids = jnp.arange(E, dtype=i32)
    perm0 = jnp.zeros((EP,), i32).at[jnp.where(is0 == 1, p0, EP - 1)].set(
        eids, mode="drop")
    perm1 = jnp.zeros((EP,), i32).at[jnp.where(is0 == 0, p1, EP - 1)].set(
        eids, mode="drop")
    ar = jnp.arange(EP, dtype=i32)

    def phase_arrays(perm, cnt):
        us = u32[perm]
        vs = v32[perm]
        msk = (ar < cnt).astype(f32).reshape(EP, 1)
        clim = jnp.full((16,), (cnt + CH - 1) // CH, i32)
        cnt1 = cnt.reshape(1).astype(i32)
        return (_stride_chunks(us, EP), _stride_chunks(vs, EP),
                us.reshape(EP // CH, CH), vs.reshape(EP // CH, CH),
                msk, clim, cnt1)

    ph0 = phase_arrays(perm0, c0)
    ph1 = phase_arrays(perm1, c1)

    nv = jnp.pad(node_vectors, ((0, NPAD - N), (0, 0)))
    nvi = jnp.pad(node_vectors_initial, ((0, NPAD - N), (0, 0)))
    zeros = jnp.zeros((NPAD // NS, D), f32)

    gather = _gather_pair(NPAD, EP, D)
    scatter = _scatter_pair(NPAD, EP, D)
    emlp = _edge_mlp(EP, D, 4 * D, 1024)
    nmlp = _node_mlp(NPAD, D, 2 * D, 1024)

    def wflat(ws):
        out = []
        for (W, b) in ws:
            out.append(W)
            out.append(b.reshape(1, -1))
        return out

    yuv = {}
    for t, ph in ((0, ph0), (1, ph1)):
        us_str, vs_str, _, _, _, clim, _ = ph
        yuv[t] = gather(nvi, us_str, vs_str, clim)

    for i in range(len(params["f_n"])):
        for ename, nname, t, ph in (("f_ef", "f_n", 0, ph0),
                                    ("f_ef2", "f_n2", 1, ph1)):
            us_str, vs_str, us_r, vs_r, msk, clim, cnt1 = ph
            xu, xv = gather(nv, us_str, vs_str, clim)
            yu, yv = yuv[t]
            mf, mr = emlp(cnt1, xu, xv, yu, yv, msk,
                          *wflat(params[ename][i]))
            aggp = scatter(mf, mr, us_r, vs_r, clim, zeros)
            nv = nmlp(aggp[0], aggp[1], nv, *wflat(params[nname][i]))

    return nv[:N]
